# baseline (speedup 1.0000x reference)
import jax
import jax.numpy as jnp
from jax.experimental import pallas as pl
from jax.experimental.pallas import tpu as pltpu

_NBUF = 3


def _fused_body(x_ref, w_ref, a_hbm, b_ref, o_ref, s_ref, bufs, sems):
    i = pl.program_id(0)
    nb = pl.num_programs(0)
    bm = bufs.shape[1]

    def _copy(j, slot):
        return pltpu.make_async_copy(
            a_hbm.at[pl.ds(j * bm, bm), :], bufs.at[slot], sems.at[slot]
        )

    @pl.when(i == 0)
    def _():
        for j in range(_NBUF - 1):
            _copy(j, j).start()
        s_ref[...] = jnp.dot(
            x_ref[...].astype(jnp.bfloat16),
            w_ref[...].astype(jnp.bfloat16),
            preferred_element_type=jnp.float32,
        ).astype(jnp.bfloat16)

    nxt = i + _NBUF - 1

    @pl.when(nxt < nb)
    def _():
        _copy(nxt, jax.lax.rem(nxt, _NBUF)).start()

    slot = jax.lax.rem(i, _NBUF)
    _copy(i, slot).wait()
    o_ref[...] = (
        jnp.dot(
            bufs[slot].astype(jnp.bfloat16),
            s_ref[...],
            preferred_element_type=jnp.float32,
        )
        + b_ref[...]
    )


def _row_block(n):
    best = 8
    for bm in range(8, 81, 8):
        if n % bm == 0:
            best = bm
    return best


def kernel(input_features, adj, W, b):
    n, d_in = input_features.shape
    d_out = W.shape[1]
    bm = _row_block(n)
    out = pl.pallas_call(
        _fused_body,
        grid=(n // bm,),
        in_specs=[
            pl.BlockSpec((n, d_in), lambda i: (0, 0)),
            pl.BlockSpec((d_in, d_out), lambda i: (0, 0)),
            pl.BlockSpec(memory_space=pltpu.MemorySpace.HBM),
            pl.BlockSpec((1, d_out), lambda i: (0, 0)),
        ],
        out_specs=pl.BlockSpec((bm, d_out), lambda i: (i, 0)),
        out_shape=jax.ShapeDtypeStruct((n, d_out), jnp.float32),
        scratch_shapes=[
            pltpu.VMEM((n, d_out), jnp.bfloat16),
            pltpu.VMEM((_NBUF, bm, n), jnp.float32),
            pltpu.SemaphoreType.DMA((_NBUF,)),
        ],
    )(input_features, W, adj, b.reshape(1, d_out))
    return out


# speedup vs baseline: 1.0161x; 1.0161x over previous
import jax
import jax.numpy as jnp
from jax.experimental import pallas as pl
from jax.experimental.pallas import tpu as pltpu

_NBUF = 4


def _fused_body(x_ref, w_ref, a_hbm, b_ref, o_ref, s_ref, bufs, sems):
    i = pl.program_id(0)
    nb = pl.num_programs(0)
    bm = bufs.shape[1]

    h = bm // 2

    def _copies(j, slot):
        return (
            pltpu.make_async_copy(
                a_hbm.at[pl.ds(j * bm, h), :],
                bufs.at[slot, pl.ds(0, h)],
                sems.at[slot, 0],
            ),
            pltpu.make_async_copy(
                a_hbm.at[pl.ds(j * bm + h, h), :],
                bufs.at[slot, pl.ds(h, h)],
                sems.at[slot, 1],
            ),
        )

    def _start(j, slot):
        c0, c1 = _copies(j, slot)
        c0.start()
        c1.start()

    @pl.when(i == 0)
    def _():
        for j in range(_NBUF - 1):
            _start(j, j)
        s_ref[...] = jnp.dot(
            x_ref[...].astype(jnp.bfloat16),
            w_ref[...].astype(jnp.bfloat16),
            preferred_element_type=jnp.float32,
        ).astype(jnp.bfloat16)

    nxt = i + _NBUF - 1

    @pl.when(nxt < nb)
    def _():
        _start(nxt, jax.lax.rem(nxt, _NBUF))

    slot = jax.lax.rem(i, _NBUF)
    c0, c1 = _copies(i, slot)
    c0.wait()
    c1.wait()
    o_ref[...] = (
        jnp.dot(
            bufs[slot].astype(jnp.bfloat16),
            s_ref[...],
            preferred_element_type=jnp.float32,
        )
        + b_ref[...]
    )


def _row_block(n):
    best = 8
    for bm in range(8, 81, 8):
        if n % bm == 0:
            best = bm
    return best


def kernel(input_features, adj, W, b):
    n, d_in = input_features.shape
    d_out = W.shape[1]
    bm = _row_block(n)
    out = pl.pallas_call(
        _fused_body,
        grid=(n // bm,),
        in_specs=[
            pl.BlockSpec((n, d_in), lambda i: (0, 0)),
            pl.BlockSpec((d_in, d_out), lambda i: (0, 0)),
            pl.BlockSpec(memory_space=pltpu.MemorySpace.HBM),
            pl.BlockSpec((1, d_out), lambda i: (0, 0)),
        ],
        out_specs=pl.BlockSpec((bm, d_out), lambda i: (i, 0)),
        out_shape=jax.ShapeDtypeStruct((n, d_out), jnp.float32),
        scratch_shapes=[
            pltpu.VMEM((n, d_out), jnp.bfloat16),
            pltpu.VMEM((_NBUF, bm, n), jnp.float32),
            pltpu.SemaphoreType.DMA((_NBUF, 2)),
        ],
    )(input_features, W, adj, b.reshape(1, d_out))
    return out


# speedup vs baseline: 1.0180x; 1.0019x over previous
import jax
import jax.numpy as jnp
from jax.experimental import pallas as pl
from jax.experimental.pallas import tpu as pltpu

_NBUF = 4


def _fused_body(x_ref, w_ref, a_hbm, b_ref, o_ref, s_ref, bufs, sems):
    i = pl.program_id(0)
    nb = pl.num_programs(0)
    bm = bufs.shape[1]

    def _copy(j, slot):
        return pltpu.make_async_copy(
            a_hbm.at[pl.ds(j * bm, bm), :], bufs.at[slot], sems.at[slot]
        )

    @pl.when(i == 0)
    def _():
        for j in range(_NBUF - 1):
            _copy(j, j).start()
        s_ref[...] = jnp.dot(
            x_ref[...].astype(jnp.bfloat16),
            w_ref[...].astype(jnp.bfloat16),
            preferred_element_type=jnp.float32,
        ).astype(jnp.bfloat16)

    nxt = i + _NBUF - 1

    @pl.when(nxt < nb)
    def _():
        _copy(nxt, jax.lax.rem(nxt, _NBUF)).start()

    slot = jax.lax.rem(i, _NBUF)
    _copy(i, slot).wait()
    o_ref[...] = (
        jnp.dot(
            bufs[slot].astype(jnp.bfloat16),
            s_ref[...],
            preferred_element_type=jnp.float32,
        )
        + b_ref[...]
    )


def _row_block(n):
    best = 8
    for bm in range(8, 81, 8):
        if n % bm == 0:
            best = bm
    return best


def kernel(input_features, adj, W, b):
    n, d_in = input_features.shape
    d_out = W.shape[1]
    bm = _row_block(n)
    out = pl.pallas_call(
        _fused_body,
        grid=(n // bm,),
        in_specs=[
            pl.BlockSpec((n, d_in), lambda i: (0, 0)),
            pl.BlockSpec((d_in, d_out), lambda i: (0, 0)),
            pl.BlockSpec(memory_space=pltpu.MemorySpace.HBM),
            pl.BlockSpec((1, d_out), lambda i: (0, 0)),
        ],
        out_specs=pl.BlockSpec((bm, d_out), lambda i: (i, 0)),
        out_shape=jax.ShapeDtypeStruct((n, d_out), jnp.float32),
        scratch_shapes=[
            pltpu.VMEM((n, d_out), jnp.bfloat16),
            pltpu.VMEM((_NBUF, bm, n), jnp.float32),
            pltpu.SemaphoreType.DMA((_NBUF,)),
        ],
    )(input_features, W, adj, b.reshape(1, d_out))
    return out
